# baseline (speedup 1.0000x reference)
import numpy as np
import jax
import jax.numpy as jnp
from jax.experimental import pallas as pl
from jax.experimental.pallas import tpu as pltpu

_B, _N, _NBRS, _D, _T = 16, 256, 16, 128, 128
_H1, _H2, _EMB, _HEADS, _HOR, _NS = 256, 256, 256, 8, 16, 1000
_HD = _EMB // _HEADS


def _pos_enc_np(length, channels):
    ch = int(np.ceil(channels / 2) * 2)
    inv_freq = 1.0 / (10000 ** (np.arange(0, ch, 2, dtype=np.float64) / ch))
    pos = np.arange(length, dtype=np.float64)
    sin_inp = np.einsum('i,j->ij', pos, inv_freq)
    emb = np.concatenate([np.sin(sin_inp), np.cos(sin_inp)], axis=-1)
    return emb[:, :channels].astype(np.float32)


def _lrelu(x):
    return jnp.maximum(x, 0.01 * x)


def _fused_body(ne_ref, tgt_ref, sn_ref, et_ref, nm_ref, trav3_ref, travc_ref,
                pe_ref,
                wt_ref, wd_ref, wsg_ref, we_ref, b1_ref,
                w2_ref, b2_ref, opw_ref,
                gwt_ref, gb1_ref, gw2_ref, gb2_ref, gopw_ref,
                bias_ref,
                qw_ref, qb_ref, kw_ref, kb_ref, vw_ref, vb_ref,
                wq_ref, bq_ref, wk_ref, bk_ref, wv_ref, bv_ref,
                ow_ref, ob_ref,
                out_ref, agg_ref, samp_ref):
    ne = ne_ref[0]
    tgt = tgt_ref[0]
    sn = sn_ref[0]
    et = et_ref[0]
    nm = nm_ref[0]

    f32 = jnp.float32
    bf16 = jnp.bfloat16
    TILE = 64
    NT = _N // TILE
    tgt_p = jnp.dot(tgt, wt_ref[...], preferred_element_type=f32)
    dst_p = jnp.dot(ne, wd_ref[...], preferred_element_type=f32).astype(bf16)
    dste = jnp.concatenate([dst_p, we_ref[...].astype(bf16)], axis=0)
    tb1 = tgt_p + b1_ref[...]
    gtb1 = (jnp.dot(tgt, gwt_ref[...], preferred_element_type=f32)
            + gb1_ref[...])
    w2b = w2_ref[...].astype(bf16)
    opwb = opw_ref[...].astype(bf16)

    for t in range(NT):
        rows = slice(t * TILE, (t + 1) * TILE)
        ne_t = ne[rows]
        sn_t = sn[rows]
        et_t = et[rows]
        nm_t = nm[rows]
        proj = jnp.dot(ne_t, wsg_ref[...], preferred_element_type=f32)

        idx = sn_t[:, :_NBRS]
        et16 = et_t[:, :_NBRS]
        iota = jax.lax.broadcasted_iota(jnp.int32, (TILE, _NBRS, _N + 2), 2)
        idx3 = idx[:, :, None]
        et3 = et16[:, :, None]
        sel_dst = idx3 == iota
        sel_e1 = (iota == _N) & (et3 == 1)
        sel_e2 = (iota == _N + 1) & (et3 == 2)
        amat = (sel_dst | sel_e1 | sel_e2).astype(bf16)
        gath = jnp.dot(amat.reshape(TILE * _NBRS, _N + 2), dste,
                       preferred_element_type=f32).astype(bf16)

        e1 = (et16 == 1).astype(f32)
        e2 = (et16 == 2).astype(f32)

        srcc = (proj[:, :_H1] + tb1).astype(bf16)
        h1 = gath.reshape(TILE, _NBRS, _H1) + srcc[:, None, :]
        h1 = _lrelu(h1)
        h2 = (jnp.dot(h1.reshape(TILE * _NBRS, _H1), w2b,
                      preferred_element_type=f32).astype(bf16)
              + b2_ref[...].astype(bf16))
        h2 = _lrelu(h2)
        prod = h2.reshape(TILE, _NBRS, _H2) * opwb[None]
        pi_e = jnp.sum(prod.astype(f32), axis=2)
        mask_e = (e1 + e2) > 0
        pi_e = jnp.where(mask_e, pi_e, 0.0)

        g1 = _lrelu(proj[:, _H1:] + gtb1)
        g2 = _lrelu(jnp.dot(g1, gw2_ref[...], preferred_element_type=f32)
                    + gb2_ref[...])
        pi_g = jnp.sum(g2 * gopw_ref[...], axis=1, keepdims=True)
        pi_g = jnp.where(nm_t == 0.0, pi_g, 0.0)

        pi = jnp.concatenate([pi_e, pi_g], axis=1)
        pi = pi + bias_ref[...]
        logits = jnp.where(et_t != 0, pi, f32(-1e30))
        mx = jnp.max(logits, axis=1, keepdims=True)
        z = jnp.exp(logits - mx)
        prob = z / jnp.sum(z, axis=1, keepdims=True)
        out_ref[0, rows, :] = jnp.log(prob + 1e-5)

    trav_row = trav3_ref[0]
    trav_col = travc_ref[0]
    iota_a = jax.lax.broadcasted_iota(jnp.int32, (_HOR, _N), 1)
    oh_a = ((trav_col == iota_a) & (trav_col < _N)).astype(f32)
    sel = jnp.dot(oh_a, ne, preferred_element_type=f32) + pe_ref[...]

    keys = jnp.dot(sel, kw_ref[...], preferred_element_type=f32) + kb_ref[...]
    vals = jnp.dot(sel, vw_ref[...], preferred_element_type=f32) + vb_ref[...]
    query = jnp.dot(tgt, qw_ref[...], preferred_element_type=f32) + qb_ref[...]

    scale = f32(1.0 / np.sqrt(_HD))
    qp = (jnp.dot(query, wq_ref[...], preferred_element_type=f32)
          + bq_ref[...]) * scale
    kp = jnp.dot(keys, wk_ref[...], preferred_element_type=f32) + bk_ref[...]
    vp = jnp.dot(vals, wv_ref[...], preferred_element_type=f32) + bv_ref[...]

    valid_col = trav_col < _N
    outs = []
    for h in range(_HEADS):
        sl = slice(h * _HD, (h + 1) * _HD)
        qh = qp[:, sl]
        kh = kp[:, sl]
        vh = vp[:, sl]
        sc = jnp.sum(qh * kh, axis=1, keepdims=True)
        sc = jnp.where(valid_col, sc, f32(-1e30))
        mxs = jnp.max(sc, axis=0, keepdims=True)
        zs = jnp.exp(sc - mxs)
        aw = zs / jnp.sum(zs, axis=0, keepdims=True)
        outs.append(jnp.sum(aw * vh, axis=0, keepdims=True))
    att = jnp.concatenate(outs, axis=1)
    att = jnp.dot(att, ow_ref[...], preferred_element_type=f32) + ob_ref[...]
    row = jnp.concatenate([tgt, att], axis=1)
    agg_ref[0] = jnp.broadcast_to(row, (_NS, _T + _EMB))
    samp_ref[0] = jnp.broadcast_to(trav_row, (_NS, _HOR))


def kernel(target_agent_encoding, node_encodings, node_masks, s_next,
           edge_type, node_seq_gt, edge_on_route_mask, node_on_route_mask,
           params):
    p = params
    f32 = jnp.float32
    tgt = target_agent_encoding.astype(f32)
    ne = node_encodings.astype(f32)
    sn = s_next.astype(jnp.int32)
    et = edge_type.astype(jnp.int32)
    trav = node_seq_gt.astype(jnp.int32)
    nm = node_masks.reshape(_B, _N, 1)

    w1 = p['pi_h1_w']
    wt = w1[:, :_T].T
    ws = w1[:, _T:_T + _D].T
    wd = w1[:, _T + _D:_T + 2 * _D].T
    we = w1[:, _T + 2 * _D:].T
    b1 = p['pi_h1_b'].reshape(1, _H1)
    w2 = p['pi_h2_w'].T
    b2 = p['pi_h2_b'].reshape(1, _H2)
    opw = p['pi_op_w']
    gw = p['pi_h1_goal_w']
    gwt = gw[:, :_T].T
    gws = gw[:, _T:].T
    gb1 = p['pi_h1_goal_b'].reshape(1, _H1)
    gw2 = p['pi_h2_goal_w'].T
    gb2 = p['pi_h2_goal_b'].reshape(1, _H2)
    gopw = p['pi_op_goal_w']
    wsg = jnp.concatenate([ws, gws], axis=1)
    bias_row = jnp.concatenate(
        [jnp.broadcast_to(p['pi_op_b'].reshape(1, 1), (1, _NBRS)),
         p['pi_op_goal_b'].reshape(1, 1)], axis=1)

    tgt3 = tgt.reshape(_B, 1, _T)
    trav3 = trav.reshape(_B, 1, _HOR)
    travc = trav.reshape(_B, _HOR, 1)

    pe = jnp.asarray(_pos_enc_np(_HOR, _D))
    in_w, in_b = p['in_w'], p['in_b']
    wq2 = in_w[:_EMB].T
    wk2 = in_w[_EMB:2 * _EMB].T
    wv2 = in_w[2 * _EMB:].T
    bq = in_b[:_EMB].reshape(1, _EMB)
    bk = in_b[_EMB:2 * _EMB].reshape(1, _EMB)
    bv = in_b[2 * _EMB:].reshape(1, _EMB)

    wfull = pl.BlockSpec(index_map=lambda b: (0, 0))
    log_pi, agg_enc, samp = pl.pallas_call(
        _fused_body,
        grid=(_B,),
        in_specs=[
            pl.BlockSpec((1, _N, _D), lambda b: (b, 0, 0)),
            pl.BlockSpec((1, 1, _T), lambda b: (b, 0, 0)),
            pl.BlockSpec((1, _N, _NBRS + 1), lambda b: (b, 0, 0)),
            pl.BlockSpec((1, _N, _NBRS + 1), lambda b: (b, 0, 0)),
            pl.BlockSpec((1, _N, 1), lambda b: (b, 0, 0)),
            pl.BlockSpec((1, 1, _HOR), lambda b: (b, 0, 0)),
            pl.BlockSpec((1, _HOR, 1), lambda b: (b, 0, 0)),
            wfull,
            wfull, wfull, wfull, wfull, wfull,
            wfull, wfull, wfull,
            wfull, wfull, wfull, wfull, wfull,
            wfull,
            wfull, wfull, wfull, wfull, wfull, wfull,
            wfull, wfull, wfull, wfull, wfull, wfull,
            wfull, wfull,
        ],
        out_specs=[
            pl.BlockSpec((1, _N, _NBRS + 1), lambda b: (b, 0, 0)),
            pl.BlockSpec((1, _NS, _T + _EMB), lambda b: (b, 0, 0)),
            pl.BlockSpec((1, _NS, _HOR), lambda b: (b, 0, 0)),
        ],
        out_shape=[
            jax.ShapeDtypeStruct((_B, _N, _NBRS + 1), f32),
            jax.ShapeDtypeStruct((_B, _NS, _T + _EMB), f32),
            jax.ShapeDtypeStruct((_B, _NS, _HOR), jnp.int32),
        ],
        compiler_params=pltpu.CompilerParams(
            dimension_semantics=("arbitrary",)),
    )(ne, tgt3, sn, et, nm, trav3, travc,
      pe,
      wt, wd, wsg, we, b1, w2, b2, opw,
      gwt, gb1, gw2, gb2, gopw, bias_row,
      p['q_w'].T, p['q_b'].reshape(1, _EMB),
      p['k_w'].T, p['k_b'].reshape(1, _EMB),
      p['v_w'].T, p['v_b'].reshape(1, _EMB),
      wq2, bq, wk2, bk, wv2, bv,
      p['out_w'].T, p['out_b'].reshape(1, _EMB))

    return agg_enc, log_pi, samp


# speedup vs baseline: 1.0024x; 1.0024x over previous
import numpy as np
import jax
import jax.numpy as jnp
from jax.experimental import pallas as pl
from jax.experimental.pallas import tpu as pltpu

_B, _N, _NBRS, _D, _T = 16, 256, 16, 128, 128
_H1, _H2, _EMB, _HEADS, _HOR, _NS = 256, 256, 256, 8, 16, 1000
_HD = _EMB // _HEADS


def _pos_enc_np(length, channels):
    ch = int(np.ceil(channels / 2) * 2)
    inv_freq = 1.0 / (10000 ** (np.arange(0, ch, 2, dtype=np.float64) / ch))
    pos = np.arange(length, dtype=np.float64)
    sin_inp = np.einsum('i,j->ij', pos, inv_freq)
    emb = np.concatenate([np.sin(sin_inp), np.cos(sin_inp)], axis=-1)
    return emb[:, :channels].astype(np.float32)


def _lrelu(x):
    return jnp.maximum(x, 0.01 * x)


def _fused_body(ne_ref, tgt_ref, sn_ref, et_ref, nm_ref, trav3_ref, travc_ref,
                pe_ref,
                wt_ref, wd_ref, wsg_ref, we_ref, b1_ref,
                w2_ref, b2_ref, opw_ref,
                gwt_ref, gb1_ref, gw2_ref, gb2_ref, gopw_ref,
                bias_ref,
                qw_ref, qb_ref, kw_ref, kb_ref, vw_ref, vb_ref,
                wq_ref, bq_ref, wk_ref, bk_ref, wv_ref, bv_ref,
                ow_ref, ob_ref,
                out_ref, agg_ref, samp_ref):
    ne = ne_ref[0]
    tgt = tgt_ref[0]
    sn = sn_ref[0]
    et = et_ref[0]
    nm = nm_ref[0]

    f32 = jnp.float32
    bf16 = jnp.bfloat16
    TILE = 128
    NT = _N // TILE
    tgt_p = jnp.dot(tgt, wt_ref[...], preferred_element_type=f32)
    dst_p = jnp.dot(ne, wd_ref[...], preferred_element_type=f32).astype(bf16)
    dste = jnp.concatenate([dst_p, we_ref[...].astype(bf16)], axis=0)
    tb1 = tgt_p + b1_ref[...]
    gtb1 = (jnp.dot(tgt, gwt_ref[...], preferred_element_type=f32)
            + gb1_ref[...])
    w2b = w2_ref[...].astype(bf16)
    opwb = opw_ref[...].astype(bf16)

    for t in range(NT):
        rows = slice(t * TILE, (t + 1) * TILE)
        ne_t = ne[rows]
        sn_t = sn[rows]
        et_t = et[rows]
        nm_t = nm[rows]
        proj = jnp.dot(ne_t, wsg_ref[...], preferred_element_type=f32)

        idx = sn_t[:, :_NBRS]
        et16 = et_t[:, :_NBRS]
        iota = jax.lax.broadcasted_iota(jnp.int32, (TILE, _NBRS, _N + 2), 2)
        idx3 = idx[:, :, None]
        et3 = et16[:, :, None]
        sel_dst = idx3 == iota
        sel_e1 = (iota == _N) & (et3 == 1)
        sel_e2 = (iota == _N + 1) & (et3 == 2)
        amat = (sel_dst | sel_e1 | sel_e2).astype(bf16)
        gath = jnp.dot(amat.reshape(TILE * _NBRS, _N + 2), dste,
                       preferred_element_type=f32).astype(bf16)

        e1 = (et16 == 1).astype(f32)
        e2 = (et16 == 2).astype(f32)

        srcc = (proj[:, :_H1] + tb1).astype(bf16)
        h1 = gath.reshape(TILE, _NBRS, _H1) + srcc[:, None, :]
        h1 = _lrelu(h1)
        h2 = (jnp.dot(h1.reshape(TILE * _NBRS, _H1), w2b,
                      preferred_element_type=f32).astype(bf16)
              + b2_ref[...].astype(bf16))
        h2 = _lrelu(h2)
        prod = h2.reshape(TILE, _NBRS, _H2) * opwb[None]
        pi_e = jnp.sum(prod.astype(f32), axis=2)
        mask_e = (e1 + e2) > 0
        pi_e = jnp.where(mask_e, pi_e, 0.0)

        g1 = _lrelu(proj[:, _H1:] + gtb1)
        g2 = _lrelu(jnp.dot(g1, gw2_ref[...], preferred_element_type=f32)
                    + gb2_ref[...])
        pi_g = jnp.sum(g2 * gopw_ref[...], axis=1, keepdims=True)
        pi_g = jnp.where(nm_t == 0.0, pi_g, 0.0)

        pi = jnp.concatenate([pi_e, pi_g], axis=1)
        pi = pi + bias_ref[...]
        logits = jnp.where(et_t != 0, pi, f32(-1e30))
        mx = jnp.max(logits, axis=1, keepdims=True)
        z = jnp.exp(logits - mx)
        prob = z / jnp.sum(z, axis=1, keepdims=True)
        out_ref[0, rows, :] = jnp.log(prob + 1e-5)

    trav_row = trav3_ref[0]
    trav_col = travc_ref[0]
    iota_a = jax.lax.broadcasted_iota(jnp.int32, (_HOR, _N), 1)
    oh_a = ((trav_col == iota_a) & (trav_col < _N)).astype(f32)
    sel = jnp.dot(oh_a, ne, preferred_element_type=f32) + pe_ref[...]

    keys = jnp.dot(sel, kw_ref[...], preferred_element_type=f32) + kb_ref[...]
    vals = jnp.dot(sel, vw_ref[...], preferred_element_type=f32) + vb_ref[...]
    query = jnp.dot(tgt, qw_ref[...], preferred_element_type=f32) + qb_ref[...]

    scale = f32(1.0 / np.sqrt(_HD))
    qp = (jnp.dot(query, wq_ref[...], preferred_element_type=f32)
          + bq_ref[...]) * scale
    kp = jnp.dot(keys, wk_ref[...], preferred_element_type=f32) + bk_ref[...]
    vp = jnp.dot(vals, wv_ref[...], preferred_element_type=f32) + bv_ref[...]

    valid_col = trav_col < _N
    outs = []
    for h in range(_HEADS):
        sl = slice(h * _HD, (h + 1) * _HD)
        qh = qp[:, sl]
        kh = kp[:, sl]
        vh = vp[:, sl]
        sc = jnp.sum(qh * kh, axis=1, keepdims=True)
        sc = jnp.where(valid_col, sc, f32(-1e30))
        mxs = jnp.max(sc, axis=0, keepdims=True)
        zs = jnp.exp(sc - mxs)
        aw = zs / jnp.sum(zs, axis=0, keepdims=True)
        outs.append(jnp.sum(aw * vh, axis=0, keepdims=True))
    att = jnp.concatenate(outs, axis=1)
    att = jnp.dot(att, ow_ref[...], preferred_element_type=f32) + ob_ref[...]
    row = jnp.concatenate([tgt, att], axis=1)
    agg_ref[0] = jnp.broadcast_to(row, (_NS, _T + _EMB))
    samp_ref[0] = jnp.broadcast_to(trav_row, (_NS, _HOR))


def kernel(target_agent_encoding, node_encodings, node_masks, s_next,
           edge_type, node_seq_gt, edge_on_route_mask, node_on_route_mask,
           params):
    p = params
    f32 = jnp.float32
    tgt = target_agent_encoding.astype(f32)
    ne = node_encodings.astype(f32)
    sn = s_next.astype(jnp.int32)
    et = edge_type.astype(jnp.int32)
    trav = node_seq_gt.astype(jnp.int32)
    nm = node_masks.reshape(_B, _N, 1)

    w1 = p['pi_h1_w']
    wt = w1[:, :_T].T
    ws = w1[:, _T:_T + _D].T
    wd = w1[:, _T + _D:_T + 2 * _D].T
    we = w1[:, _T + 2 * _D:].T
    b1 = p['pi_h1_b'].reshape(1, _H1)
    w2 = p['pi_h2_w'].T
    b2 = p['pi_h2_b'].reshape(1, _H2)
    opw = p['pi_op_w']
    gw = p['pi_h1_goal_w']
    gwt = gw[:, :_T].T
    gws = gw[:, _T:].T
    gb1 = p['pi_h1_goal_b'].reshape(1, _H1)
    gw2 = p['pi_h2_goal_w'].T
    gb2 = p['pi_h2_goal_b'].reshape(1, _H2)
    gopw = p['pi_op_goal_w']
    wsg = jnp.concatenate([ws, gws], axis=1)
    bias_row = jnp.concatenate(
        [jnp.broadcast_to(p['pi_op_b'].reshape(1, 1), (1, _NBRS)),
         p['pi_op_goal_b'].reshape(1, 1)], axis=1)

    tgt3 = tgt.reshape(_B, 1, _T)
    trav3 = trav.reshape(_B, 1, _HOR)
    travc = trav.reshape(_B, _HOR, 1)

    pe = jnp.asarray(_pos_enc_np(_HOR, _D))
    in_w, in_b = p['in_w'], p['in_b']
    wq2 = in_w[:_EMB].T
    wk2 = in_w[_EMB:2 * _EMB].T
    wv2 = in_w[2 * _EMB:].T
    bq = in_b[:_EMB].reshape(1, _EMB)
    bk = in_b[_EMB:2 * _EMB].reshape(1, _EMB)
    bv = in_b[2 * _EMB:].reshape(1, _EMB)

    wfull = pl.BlockSpec(index_map=lambda b: (0, 0))
    log_pi, agg_enc, samp = pl.pallas_call(
        _fused_body,
        grid=(_B,),
        in_specs=[
            pl.BlockSpec((1, _N, _D), lambda b: (b, 0, 0)),
            pl.BlockSpec((1, 1, _T), lambda b: (b, 0, 0)),
            pl.BlockSpec((1, _N, _NBRS + 1), lambda b: (b, 0, 0)),
            pl.BlockSpec((1, _N, _NBRS + 1), lambda b: (b, 0, 0)),
            pl.BlockSpec((1, _N, 1), lambda b: (b, 0, 0)),
            pl.BlockSpec((1, 1, _HOR), lambda b: (b, 0, 0)),
            pl.BlockSpec((1, _HOR, 1), lambda b: (b, 0, 0)),
            wfull,
            wfull, wfull, wfull, wfull, wfull,
            wfull, wfull, wfull,
            wfull, wfull, wfull, wfull, wfull,
            wfull,
            wfull, wfull, wfull, wfull, wfull, wfull,
            wfull, wfull, wfull, wfull, wfull, wfull,
            wfull, wfull,
        ],
        out_specs=[
            pl.BlockSpec((1, _N, _NBRS + 1), lambda b: (b, 0, 0)),
            pl.BlockSpec((1, _NS, _T + _EMB), lambda b: (b, 0, 0)),
            pl.BlockSpec((1, _NS, _HOR), lambda b: (b, 0, 0)),
        ],
        out_shape=[
            jax.ShapeDtypeStruct((_B, _N, _NBRS + 1), f32),
            jax.ShapeDtypeStruct((_B, _NS, _T + _EMB), f32),
            jax.ShapeDtypeStruct((_B, _NS, _HOR), jnp.int32),
        ],
        compiler_params=pltpu.CompilerParams(
            dimension_semantics=("arbitrary",),
            vmem_limit_bytes=100 * 1024 * 1024),
    )(ne, tgt3, sn, et, nm, trav3, travc,
      pe,
      wt, wd, wsg, we, b1, w2, b2, opw,
      gwt, gb1, gw2, gb2, gopw, bias_row,
      p['q_w'].T, p['q_b'].reshape(1, _EMB),
      p['k_w'].T, p['k_b'].reshape(1, _EMB),
      p['v_w'].T, p['v_b'].reshape(1, _EMB),
      wq2, bq, wk2, bk, wv2, bv,
      p['out_w'].T, p['out_b'].reshape(1, _EMB))

    return agg_enc, log_pi, samp


# speedup vs baseline: 1.3559x; 1.3527x over previous
import numpy as np
import jax
import jax.numpy as jnp
from jax.experimental import pallas as pl
from jax.experimental.pallas import tpu as pltpu

_B, _N, _NBRS, _D, _T = 16, 256, 16, 128, 128
_H1, _H2, _EMB, _HEADS, _HOR, _NS = 256, 256, 256, 8, 16, 1000
_HD = _EMB // _HEADS


def _pos_enc_np(length, channels):
    ch = int(np.ceil(channels / 2) * 2)
    inv_freq = 1.0 / (10000 ** (np.arange(0, ch, 2, dtype=np.float64) / ch))
    pos = np.arange(length, dtype=np.float64)
    sin_inp = np.einsum('i,j->ij', pos, inv_freq)
    emb = np.concatenate([np.sin(sin_inp), np.cos(sin_inp)], axis=-1)
    return emb[:, :channels].astype(np.float32)


def _lrelu(x):
    return jnp.maximum(x, 0.01 * x)


def _fused_body(ne_ref, tgt_ref, sn_ref, et_ref, nm_ref, trav3_ref, travc_ref,
                pe_ref,
                wt_ref, wd_ref, wsg_ref, we_ref, b1_ref,
                w2_ref, b2_ref, opw_ref,
                gwt_ref, gb1_ref, gw2_ref, gb2_ref, gopw_ref,
                bias_ref,
                qw_ref, qb_ref, kw_ref, kb_ref, vw_ref, vb_ref,
                wq_ref, bq_ref, wk_ref, bk_ref, wv_ref, bv_ref,
                ow_ref, ob_ref,
                out_ref, agg_ref, samp_ref):
    ne = ne_ref[0]
    tgt = tgt_ref[0]
    sn = sn_ref[0]
    et = et_ref[0]
    nm = nm_ref[0]

    f32 = jnp.float32
    bf16 = jnp.bfloat16
    TILE = 256
    NT = _N // TILE
    tgt_p = jnp.dot(tgt, wt_ref[...], preferred_element_type=f32)
    dst_p = jnp.dot(ne, wd_ref[...], preferred_element_type=f32).astype(bf16)
    web = we_ref[...].astype(bf16)
    tb1 = tgt_p + b1_ref[...]
    gtb1 = (jnp.dot(tgt, gwt_ref[...], preferred_element_type=f32)
            + gb1_ref[...])
    w2b = w2_ref[...].astype(bf16)
    opwb = opw_ref[...].astype(bf16)

    for t in range(NT):
        rows = slice(t * TILE, (t + 1) * TILE)
        ne_t = ne[rows]
        sn_t = sn[rows]
        et_t = et[rows]
        nm_t = nm[rows]
        proj = jnp.dot(ne_t, wsg_ref[...], preferred_element_type=f32)

        idx = sn_t[:, :_NBRS]
        et16 = et_t[:, :_NBRS]
        iota = jax.lax.broadcasted_iota(jnp.int32, (TILE, _NBRS, _N), 2)
        idx3 = idx[:, :, None]
        amat = (idx3 == iota).astype(bf16)
        gath = jnp.dot(amat.reshape(TILE * _NBRS, _N), dst_p,
                       preferred_element_type=f32).astype(bf16)

        e1 = (et16 == 1).astype(f32)
        e2 = (et16 == 2).astype(f32)
        e1b = (et16 == 1).astype(bf16)[:, :, None]
        e2b = (et16 == 2).astype(bf16)[:, :, None]
        edge_p = e1b * web[0:1, :][None] + e2b * web[1:2, :][None]

        srcc = (proj[:, :_H1] + tb1).astype(bf16)
        h1 = gath.reshape(TILE, _NBRS, _H1) + srcc[:, None, :] + edge_p
        h1 = _lrelu(h1)
        h2 = (jnp.dot(h1.reshape(TILE * _NBRS, _H1), w2b,
                      preferred_element_type=f32).astype(bf16)
              + b2_ref[...].astype(bf16))
        h2 = _lrelu(h2)
        prod = h2.reshape(TILE, _NBRS, _H2) * opwb[None]
        pi_e = jnp.sum(prod.astype(f32), axis=2)
        mask_e = (e1 + e2) > 0
        pi_e = jnp.where(mask_e, pi_e, 0.0)

        g1 = _lrelu(proj[:, _H1:] + gtb1)
        g2 = _lrelu(jnp.dot(g1, gw2_ref[...], preferred_element_type=f32)
                    + gb2_ref[...])
        pi_g = jnp.sum(g2 * gopw_ref[...], axis=1, keepdims=True)
        pi_g = jnp.where(nm_t == 0.0, pi_g, 0.0)

        pi = jnp.concatenate([pi_e, pi_g], axis=1)
        pi = pi + bias_ref[...]
        logits = jnp.where(et_t != 0, pi, f32(-1e30))
        mx = jnp.max(logits, axis=1, keepdims=True)
        z = jnp.exp(logits - mx)
        prob = z / jnp.sum(z, axis=1, keepdims=True)
        out_ref[0, rows, :] = jnp.log(prob + 1e-5)

    trav_row = trav3_ref[0]
    trav_col = travc_ref[0]
    iota_a = jax.lax.broadcasted_iota(jnp.int32, (_HOR, _N), 1)
    oh_a = ((trav_col == iota_a) & (trav_col < _N)).astype(f32)
    sel = jnp.dot(oh_a, ne, preferred_element_type=f32) + pe_ref[...]

    keys = jnp.dot(sel, kw_ref[...], preferred_element_type=f32) + kb_ref[...]
    vals = jnp.dot(sel, vw_ref[...], preferred_element_type=f32) + vb_ref[...]
    query = jnp.dot(tgt, qw_ref[...], preferred_element_type=f32) + qb_ref[...]

    scale = f32(1.0 / np.sqrt(_HD))
    qp = (jnp.dot(query, wq_ref[...], preferred_element_type=f32)
          + bq_ref[...]) * scale
    kp = jnp.dot(keys, wk_ref[...], preferred_element_type=f32) + bk_ref[...]
    vp = jnp.dot(vals, wv_ref[...], preferred_element_type=f32) + bv_ref[...]

    valid_col = trav_col < _N
    outs = []
    for h in range(_HEADS):
        sl = slice(h * _HD, (h + 1) * _HD)
        qh = qp[:, sl]
        kh = kp[:, sl]
        vh = vp[:, sl]
        sc = jnp.sum(qh * kh, axis=1, keepdims=True)
        sc = jnp.where(valid_col, sc, f32(-1e30))
        mxs = jnp.max(sc, axis=0, keepdims=True)
        zs = jnp.exp(sc - mxs)
        aw = zs / jnp.sum(zs, axis=0, keepdims=True)
        outs.append(jnp.sum(aw * vh, axis=0, keepdims=True))
    att = jnp.concatenate(outs, axis=1)
    att = jnp.dot(att, ow_ref[...], preferred_element_type=f32) + ob_ref[...]
    row = jnp.concatenate([tgt, att], axis=1)
    agg_ref[0] = jnp.broadcast_to(row, (_NS, _T + _EMB))
    samp_ref[0] = jnp.broadcast_to(trav_row, (_NS, _HOR))


def kernel(target_agent_encoding, node_encodings, node_masks, s_next,
           edge_type, node_seq_gt, edge_on_route_mask, node_on_route_mask,
           params):
    p = params
    f32 = jnp.float32
    tgt = target_agent_encoding.astype(f32)
    ne = node_encodings.astype(f32)
    sn = s_next.astype(jnp.int32)
    et = edge_type.astype(jnp.int32)
    trav = node_seq_gt.astype(jnp.int32)
    nm = node_masks.reshape(_B, _N, 1)

    w1 = p['pi_h1_w']
    wt = w1[:, :_T].T
    ws = w1[:, _T:_T + _D].T
    wd = w1[:, _T + _D:_T + 2 * _D].T
    we = w1[:, _T + 2 * _D:].T
    b1 = p['pi_h1_b'].reshape(1, _H1)
    w2 = p['pi_h2_w'].T
    b2 = p['pi_h2_b'].reshape(1, _H2)
    opw = p['pi_op_w']
    gw = p['pi_h1_goal_w']
    gwt = gw[:, :_T].T
    gws = gw[:, _T:].T
    gb1 = p['pi_h1_goal_b'].reshape(1, _H1)
    gw2 = p['pi_h2_goal_w'].T
    gb2 = p['pi_h2_goal_b'].reshape(1, _H2)
    gopw = p['pi_op_goal_w']
    wsg = jnp.concatenate([ws, gws], axis=1)
    bias_row = jnp.concatenate(
        [jnp.broadcast_to(p['pi_op_b'].reshape(1, 1), (1, _NBRS)),
         p['pi_op_goal_b'].reshape(1, 1)], axis=1)

    tgt3 = tgt.reshape(_B, 1, _T)
    trav3 = trav.reshape(_B, 1, _HOR)
    travc = trav.reshape(_B, _HOR, 1)

    pe = jnp.asarray(_pos_enc_np(_HOR, _D))
    in_w, in_b = p['in_w'], p['in_b']
    wq2 = in_w[:_EMB].T
    wk2 = in_w[_EMB:2 * _EMB].T
    wv2 = in_w[2 * _EMB:].T
    bq = in_b[:_EMB].reshape(1, _EMB)
    bk = in_b[_EMB:2 * _EMB].reshape(1, _EMB)
    bv = in_b[2 * _EMB:].reshape(1, _EMB)

    wfull = pl.BlockSpec(index_map=lambda b: (0, 0))
    log_pi, agg_enc, samp = pl.pallas_call(
        _fused_body,
        grid=(_B,),
        in_specs=[
            pl.BlockSpec((1, _N, _D), lambda b: (b, 0, 0)),
            pl.BlockSpec((1, 1, _T), lambda b: (b, 0, 0)),
            pl.BlockSpec((1, _N, _NBRS + 1), lambda b: (b, 0, 0)),
            pl.BlockSpec((1, _N, _NBRS + 1), lambda b: (b, 0, 0)),
            pl.BlockSpec((1, _N, 1), lambda b: (b, 0, 0)),
            pl.BlockSpec((1, 1, _HOR), lambda b: (b, 0, 0)),
            pl.BlockSpec((1, _HOR, 1), lambda b: (b, 0, 0)),
            wfull,
            wfull, wfull, wfull, wfull, wfull,
            wfull, wfull, wfull,
            wfull, wfull, wfull, wfull, wfull,
            wfull,
            wfull, wfull, wfull, wfull, wfull, wfull,
            wfull, wfull, wfull, wfull, wfull, wfull,
            wfull, wfull,
        ],
        out_specs=[
            pl.BlockSpec((1, _N, _NBRS + 1), lambda b: (b, 0, 0)),
            pl.BlockSpec((1, _NS, _T + _EMB), lambda b: (b, 0, 0)),
            pl.BlockSpec((1, _NS, _HOR), lambda b: (b, 0, 0)),
        ],
        out_shape=[
            jax.ShapeDtypeStruct((_B, _N, _NBRS + 1), f32),
            jax.ShapeDtypeStruct((_B, _NS, _T + _EMB), f32),
            jax.ShapeDtypeStruct((_B, _NS, _HOR), jnp.int32),
        ],
        compiler_params=pltpu.CompilerParams(
            dimension_semantics=("arbitrary",),
            vmem_limit_bytes=100 * 1024 * 1024),
    )(ne, tgt3, sn, et, nm, trav3, travc,
      pe,
      wt, wd, wsg, we, b1, w2, b2, opw,
      gwt, gb1, gw2, gb2, gopw, bias_row,
      p['q_w'].T, p['q_b'].reshape(1, _EMB),
      p['k_w'].T, p['k_b'].reshape(1, _EMB),
      p['v_w'].T, p['v_b'].reshape(1, _EMB),
      wq2, bq, wk2, bk, wv2, bv,
      p['out_w'].T, p['out_b'].reshape(1, _EMB))

    return agg_enc, log_pi, samp
